# baseline (speedup 1.0000x reference)
import functools
import math

import jax
import jax.numpy as jnp
from jax.experimental import pallas as pl
from jax.experimental.pallas import tpu as pltpu
from jax.experimental.pallas import tpu_sc as plsc

B, S, D = 1, 2048, 1024
H, KVH, DH = 16, 16, 64
E, K, F = 8, 2, 2048
EPS = 1e-06
THETA = 10000.0
RES_SCALE = 1.4 / math.sqrt(40.0)

SB = 256
NSB = S // SB
M = 256
NT = 24
XROWS = NT * M

_INTERPRET = False


def _qkv_body(pos_ref, x_ref, ln_ref, wq_ref, wk_ref, wv_ref,
              q_ref, k_ref, v_ref):
    x = x_ref[...]
    var = jnp.mean(x * x, axis=-1, keepdims=True)
    xn = (ln_ref[...] * (x * jax.lax.rsqrt(var + EPS))).astype(jnp.bfloat16)
    q = jnp.dot(xn, wq_ref[...], preferred_element_type=jnp.float32)
    k = jnp.dot(xn, wk_ref[...], preferred_element_type=jnp.float32)
    v = jnp.dot(xn, wv_ref[...], preferred_element_type=jnp.float32)

    pos = pos_ref[...].astype(jnp.float32)
    i2 = jax.lax.broadcasted_iota(
        jnp.int32, (1, DH // 2), 1).astype(jnp.float32) * 2.0
    inv_freq = jnp.exp(-(i2 / DH) * math.log(THETA))
    ang = pos * inv_freq
    c = jnp.cos(ang)
    sn = jnp.sin(ang)
    cos = jnp.concatenate([c, c], axis=-1)
    sin = jnp.concatenate([sn, sn], axis=-1)

    hw = DH // 2
    for h in range(H):
        qh = q[:, h * DH:(h + 1) * DH]
        qr = jnp.concatenate([-qh[:, hw:], qh[:, :hw]], axis=-1)
        q_ref[h, :, :] = (qh * cos + qr * sin).astype(jnp.bfloat16)
        kh = k[:, h * DH:(h + 1) * DH]
        kr = jnp.concatenate([-kh[:, hw:], kh[:, :hw]], axis=-1)
        k_ref[h, :, :] = (kh * cos + kr * sin).astype(jnp.bfloat16)
        v_ref[h, :, :] = v[:, h * DH:(h + 1) * DH].astype(jnp.bfloat16)


def _qkv_call(pos, x, ln1, Wq, Wk, Wv):
    return pl.pallas_call(
        _qkv_body,
        grid=(NSB,),
        in_specs=[
            pl.BlockSpec((SB, 1), lambda i: (i, 0)),
            pl.BlockSpec((SB, D), lambda i: (i, 0)),
            pl.BlockSpec((1, D), lambda i: (0, 0)),
            pl.BlockSpec((D, H * DH), lambda i: (0, 0)),
            pl.BlockSpec((D, KVH * DH), lambda i: (0, 0)),
            pl.BlockSpec((D, KVH * DH), lambda i: (0, 0)),
        ],
        out_specs=[
            pl.BlockSpec((H, SB, DH), lambda i: (0, i, 0)),
            pl.BlockSpec((KVH, SB, DH), lambda i: (0, i, 0)),
            pl.BlockSpec((KVH, SB, DH), lambda i: (0, i, 0)),
        ],
        out_shape=[
            jax.ShapeDtypeStruct((H, S, DH), jnp.bfloat16),
            jax.ShapeDtypeStruct((KVH, S, DH), jnp.bfloat16),
            jax.ShapeDtypeStruct((KVH, S, DH), jnp.bfloat16),
        ],
        interpret=_INTERPRET,
    )(pos, x, ln1, Wq, Wk, Wv)


def _attn_body(q_ref, k_ref, v_ref, o_ref):
    qb = pl.program_id(1)
    s = jax.lax.dot_general(
        q_ref[0], k_ref[0], (((1,), (1,)), ((), ())),
        preferred_element_type=jnp.float32) * (1.0 / math.sqrt(DH))
    qpos = qb * SB + jax.lax.broadcasted_iota(jnp.int32, (SB, S), 0)
    kpos = jax.lax.broadcasted_iota(jnp.int32, (SB, S), 1)
    s = jnp.where(kpos <= qpos, s, jnp.float32(-1e9))
    m = jnp.max(s, axis=-1, keepdims=True)
    p = jnp.exp(s - m)
    p = (p / jnp.sum(p, axis=-1, keepdims=True)).astype(jnp.bfloat16)
    o_ref[0] = jnp.dot(p, v_ref[0],
                       preferred_element_type=jnp.float32).astype(jnp.bfloat16)


def _attn_call(q, k, v):
    return pl.pallas_call(
        _attn_body,
        grid=(H, NSB),
        in_specs=[
            pl.BlockSpec((1, SB, DH), lambda h, qb: (h, qb, 0)),
            pl.BlockSpec((1, S, DH), lambda h, qb: (h, 0, 0)),
            pl.BlockSpec((1, S, DH), lambda h, qb: (h, 0, 0)),
        ],
        out_specs=pl.BlockSpec((1, SB, DH), lambda h, qb: (h, qb, 0)),
        out_shape=jax.ShapeDtypeStruct((H, S, DH), jnp.bfloat16),
        interpret=_INTERPRET,
    )(q, k, v)


def _plan_body(o_ref, wo_ref, res_ref, ln_ref, wr_ref, h_ref, x2_ref,
               posA_ref, posB_ref, wA_ref, wB_ref, te_ref, tv_ref, xo_ref):
    for hh in range(H):
        xo_ref[:, hh * DH:(hh + 1) * DH] = o_ref[hh]
    x = res_ref[...] + jnp.dot(
        xo_ref[...], wo_ref[...],
        preferred_element_type=jnp.float32) * RES_SCALE
    h_ref[...] = x
    var = jnp.mean(x * x, axis=-1, keepdims=True)
    xn = ln_ref[...] * (x * jax.lax.rsqrt(var + EPS))
    x2_ref[...] = xn.astype(jnp.bfloat16)

    logits = jnp.dot(xn, wr_ref[...], preferred_element_type=jnp.float32)
    eio = jax.lax.broadcasted_iota(jnp.int32, (S, 128), 1)
    logits = jnp.where(eio < E, logits, jnp.float32(-1e30))
    l0 = jnp.max(logits, axis=-1, keepdims=True)
    a0 = jnp.min(jnp.where(logits == l0, eio, E), axis=-1, keepdims=True)
    lm = jnp.where(eio == a0, jnp.float32(-1e30), logits)
    l1 = jnp.max(lm, axis=-1, keepdims=True)
    a1 = jnp.min(jnp.where(lm == l1, eio, E), axis=-1, keepdims=True)
    w0 = jax.nn.sigmoid(l0 - l1)
    wA_ref[...] = w0
    wB_ref[...] = 1.0 - w0

    m0 = (eio == a0).astype(jnp.float32)
    m1 = (eio == a1).astype(jnp.float32)

    li = jax.lax.broadcasted_iota(jnp.int32, (128, 128), 0)
    lj = jax.lax.broadcasted_iota(jnp.int32, (128, 128), 1)
    tril_s = (lj < li).astype(jnp.float32)
    triu_s = (li < lj).astype(jnp.float32)

    def _col_cumsum(m, acc):
        parts = []
        for cidx in range(S // 128):
            ch = m[cidx * 128:(cidx + 1) * 128, :]
            parts.append(
                jnp.dot(tril_s, ch, preferred_element_type=jnp.float32) + acc)
            acc = acc + jnp.sum(ch, axis=0, keepdims=True)
        return jnp.concatenate(parts, axis=0), acc

    zero = jnp.zeros((1, 128), jnp.float32)
    cA, accA = _col_cumsum(m0, zero)
    cB, accB = _col_cumsum(m1, accA)
    counts = accB

    tiles = jnp.floor((counts + (M - 1)) * (1.0 / M))
    stt = jnp.dot(tiles, triu_s, preferred_element_type=jnp.float32)
    start_rows = stt * M

    posA_ref[...] = jnp.sum((start_rows + cA) * m0, axis=-1,
                            keepdims=True).astype(jnp.int32)
    posB_ref[...] = jnp.sum((start_rows + cB) * m1, axis=-1,
                            keepdims=True).astype(jnp.int32)

    jio = jax.lax.broadcasted_iota(
        jnp.int32, (128, 128), 0).astype(jnp.float32)
    sttb = jnp.broadcast_to(stt, (128, 128))
    tilb = jnp.broadcast_to(tiles, (128, 128))
    memb = jnp.logical_and(jio >= sttb, jio < sttb + tilb)
    eio_l = jax.lax.broadcasted_iota(jnp.int32, (128, 128), 1)
    te = jnp.sum(jnp.where(memb, eio_l, 0), axis=-1, keepdims=True)
    tv = (jnp.sum(memb.astype(jnp.int32), axis=-1, keepdims=True) > 0)
    te_ref[...] = jnp.where(tv, te, E - 1)
    tv_ref[...] = tv.astype(jnp.int32)


def _plan_call(o, Wo2d, res, ln2, Wr_pad):
    return pl.pallas_call(
        _plan_body,
        grid=(1,),
        in_specs=[
            pl.BlockSpec((H, S, DH), lambda i: (0, 0, 0)),
            pl.BlockSpec((H * DH, D), lambda i: (0, 0)),
            pl.BlockSpec((S, D), lambda i: (0, 0)),
            pl.BlockSpec((1, D), lambda i: (0, 0)),
            pl.BlockSpec((D, 128), lambda i: (0, 0)),
        ],
        out_specs=[
            pl.BlockSpec((S, D), lambda i: (0, 0)),
            pl.BlockSpec((S, D), lambda i: (0, 0)),
            pl.BlockSpec((S, 1), lambda i: (0, 0)),
            pl.BlockSpec((S, 1), lambda i: (0, 0)),
            pl.BlockSpec((S, 1), lambda i: (0, 0)),
            pl.BlockSpec((S, 1), lambda i: (0, 0)),
            pl.BlockSpec((128, 1), lambda i: (0, 0)),
            pl.BlockSpec((128, 1), lambda i: (0, 0)),
        ],
        out_shape=[
            jax.ShapeDtypeStruct((S, D), jnp.float32),
            jax.ShapeDtypeStruct((S, D), jnp.bfloat16),
            jax.ShapeDtypeStruct((S, 1), jnp.int32),
            jax.ShapeDtypeStruct((S, 1), jnp.int32),
            jax.ShapeDtypeStruct((S, 1), jnp.float32),
            jax.ShapeDtypeStruct((S, 1), jnp.float32),
            jax.ShapeDtypeStruct((128, 1), jnp.int32),
            jax.ShapeDtypeStruct((128, 1), jnp.int32),
        ],
        scratch_shapes=[pltpu.VMEM((S, H * DH), jnp.bfloat16)],
        interpret=_INTERPRET,
    )(o, Wo2d, res, ln2, Wr_pad)


_NW = 32
_CHUNK = S // _NW
DI = D // 2


def _dispatch_body(x2_hbm, posA_hbm, posB_hbm, xs_hbm,
                   idxA_v, idxB_v, rows_v, semA, semB):
    wid = jax.lax.axis_index("s") * 2 + jax.lax.axis_index("c")
    base = wid * _CHUNK
    pltpu.sync_copy(posA_hbm.at[pl.ds(base, _CHUNK)], idxA_v)
    pltpu.sync_copy(posB_hbm.at[pl.ds(base, _CHUNK)], idxB_v)
    pltpu.sync_copy(x2_hbm.at[pl.ds(base, _CHUNK)], rows_v)
    cpA = pltpu.async_copy(rows_v, xs_hbm.at[idxA_v], semA)
    cpB = pltpu.async_copy(rows_v, xs_hbm.at[idxB_v], semB)
    cpA.wait()
    cpB.wait()


def _dispatch_call(x2, posA, posB):
    mesh = plsc.VectorSubcoreMesh(core_axis_name="c", subcore_axis_name="s")
    return pl.kernel(
        _dispatch_body,
        mesh=mesh,
        out_type=jax.ShapeDtypeStruct((XROWS, DI), jnp.int32),
        scratch_types=[
            pltpu.VMEM((_CHUNK,), jnp.int32),
            pltpu.VMEM((_CHUNK,), jnp.int32),
            pltpu.VMEM((_CHUNK, DI), jnp.int32),
            pltpu.SemaphoreType.DMA,
            pltpu.SemaphoreType.DMA,
        ],
    )(x2, posA, posB)


def _gmm_body(te_ref, tv_ref, x_ref, wg_ref, wu_ref, wd_ref, y_ref):
    i = pl.program_id(0)

    @pl.when(tv_ref[i] == 1)
    def _():
        x = x_ref[...]
        g = jnp.dot(x, wg_ref[0], preferred_element_type=jnp.float32)
        u = jnp.dot(x, wu_ref[0], preferred_element_type=jnp.float32)
        a = (g * jax.nn.sigmoid(g) * u).astype(jnp.bfloat16)
        y_ref[...] = jnp.dot(
            a, wd_ref[0],
            preferred_element_type=jnp.float32).astype(jnp.bfloat16)


def _gmm_call(te, tv, xs, Wg, Wu, Wd):
    grid_spec = pltpu.PrefetchScalarGridSpec(
        num_scalar_prefetch=2,
        grid=(NT,),
        in_specs=[
            pl.BlockSpec((M, D), lambda i, te, tv: (i, 0)),
            pl.BlockSpec((1, D, F), lambda i, te, tv: (te[i], 0, 0)),
            pl.BlockSpec((1, D, F), lambda i, te, tv: (te[i], 0, 0)),
            pl.BlockSpec((1, F, D), lambda i, te, tv: (te[i], 0, 0)),
        ],
        out_specs=pl.BlockSpec((M, D), lambda i, te, tv: (i, 0)),
    )
    return pl.pallas_call(
        _gmm_body,
        grid_spec=grid_spec,
        out_shape=jax.ShapeDtypeStruct((XROWS, D), jnp.bfloat16),
        interpret=_INTERPRET,
    )(te, tv, xs, Wg, Wu, Wd)


def _mgather_body(ys_hbm, posA_hbm, posB_hbm, yA_hbm, yB_hbm,
                  idx_v, buf_v, sem):
    wid = jax.lax.axis_index("s") * 2 + jax.lax.axis_index("c")
    base = wid * _CHUNK
    pltpu.sync_copy(posA_hbm.at[pl.ds(base, _CHUNK)], idx_v)
    pltpu.async_copy(ys_hbm.at[idx_v], buf_v, sem).wait()
    pltpu.sync_copy(buf_v, yA_hbm.at[pl.ds(base, _CHUNK)])
    pltpu.sync_copy(posB_hbm.at[pl.ds(base, _CHUNK)], idx_v)
    pltpu.async_copy(ys_hbm.at[idx_v], buf_v, sem).wait()
    pltpu.sync_copy(buf_v, yB_hbm.at[pl.ds(base, _CHUNK)])


def _mgather_call(ys, posA, posB):
    mesh = plsc.VectorSubcoreMesh(core_axis_name="c", subcore_axis_name="s")
    return pl.kernel(
        _mgather_body,
        mesh=mesh,
        out_type=[
            jax.ShapeDtypeStruct((S, DI), jnp.int32),
            jax.ShapeDtypeStruct((S, DI), jnp.int32),
        ],
        scratch_types=[
            pltpu.VMEM((_CHUNK,), jnp.int32),
            pltpu.VMEM((_CHUNK, DI), jnp.int32),
            pltpu.SemaphoreType.DMA,
        ],
    )(ys, posA, posB)


def _mix_body(h_ref, yA_ref, yB_ref, wA_ref, wB_ref, o_ref):
    o_ref[...] = h_ref[...] + RES_SCALE * (
        wA_ref[...] * yA_ref[...].astype(jnp.float32)
        + wB_ref[...] * yB_ref[...].astype(jnp.float32))


def _mix_call(h, yA, yB, wA, wB):
    return pl.pallas_call(
        _mix_body,
        grid=(NSB,),
        in_specs=[
            pl.BlockSpec((SB, D), lambda i: (i, 0)),
            pl.BlockSpec((SB, D), lambda i: (i, 0)),
            pl.BlockSpec((SB, D), lambda i: (i, 0)),
            pl.BlockSpec((SB, 1), lambda i: (i, 0)),
            pl.BlockSpec((SB, 1), lambda i: (i, 0)),
        ],
        out_specs=pl.BlockSpec((SB, D), lambda i: (i, 0)),
        out_shape=jax.ShapeDtypeStruct((S, D), jnp.float32),
        interpret=_INTERPRET,
    )(h, yA, yB, wA, wB)


@jax.jit
def _run(hidden_states, position_ids, ln1_w, ln2_w,
         Wq, Wk, Wv, Wo, Wr, Wg, Wu, Wd):
    x = hidden_states.reshape(S, D)
    pos = position_ids.reshape(S, 1)
    ln1 = ln1_w.reshape(1, D)
    ln2 = ln2_w.reshape(1, D)
    Wr_pad = jnp.pad(Wr, ((0, 0), (0, 128 - E)))

    bf = jnp.bfloat16
    q, k, v = _qkv_call(pos, x, ln1, Wq.astype(bf), Wk.astype(bf),
                        Wv.astype(bf))
    o = _attn_call(q, k, v)
    h, x2, posA, posB, wA, wB, te, tv = _plan_call(
        o, Wo.astype(bf), x, ln2, Wr_pad)
    te = te.reshape(128)
    tv = tv.reshape(128)
    posAf = posA.reshape(S)
    posBf = posB.reshape(S)
    x2i = jax.lax.bitcast_convert_type(
        x2.reshape(S, DI, 2), jnp.int32)
    xsi = _dispatch_call(x2i, posAf, posBf)
    xs = jax.lax.bitcast_convert_type(xsi, bf).reshape(XROWS, D)
    ys = _gmm_call(te, tv, xs, Wg.astype(bf), Wu.astype(bf), Wd.astype(bf))
    ysi = jax.lax.bitcast_convert_type(
        ys.reshape(XROWS, DI, 2), jnp.int32)
    yAi, yBi = _mgather_call(ysi, posAf, posBf)
    yA = jax.lax.bitcast_convert_type(yAi, bf).reshape(S, D)
    yB = jax.lax.bitcast_convert_type(yBi, bf).reshape(S, D)
    out = _mix_call(h, yA, yB, wA, wB)
    return out.reshape(B, S, D)


def kernel(hidden_states, position_ids, ln1_w, ln2_w,
           Wq, Wk, Wv, Wo, Wr, Wg, Wu, Wd):
    return _run(hidden_states, position_ids, ln1_w, ln2_w,
                Wq, Wk, Wv, Wo, Wr, Wg, Wu, Wd)


# speedup vs baseline: 1.7960x; 1.7960x over previous
import functools
import math

import jax
import jax.numpy as jnp
from jax.experimental import pallas as pl
from jax.experimental.pallas import tpu as pltpu
from jax.experimental.pallas import tpu_sc as plsc

B, S, D = 1, 2048, 1024
H, KVH, DH = 16, 16, 64
E, K, F = 8, 2, 2048
EPS = 1e-06
THETA = 10000.0
RES_SCALE = 1.4 / math.sqrt(40.0)

SB = 256
NSB = S // SB
M = 256
NT = 24
XROWS = NT * M

_INTERPRET = False


def _qkv_body(pos_ref, x_ref, ln_ref, wq_ref, wk_ref, wv_ref,
              q_ref, k_ref, v_ref):
    x = x_ref[...]
    var = jnp.mean(x * x, axis=-1, keepdims=True)
    xn = (ln_ref[...] * (x * jax.lax.rsqrt(var + EPS))).astype(jnp.bfloat16)
    q = jnp.dot(xn, wq_ref[...], preferred_element_type=jnp.float32)
    k = jnp.dot(xn, wk_ref[...], preferred_element_type=jnp.float32)
    v = jnp.dot(xn, wv_ref[...], preferred_element_type=jnp.float32)

    pos = pos_ref[...].astype(jnp.float32)
    i2 = jax.lax.broadcasted_iota(
        jnp.int32, (1, DH // 2), 1).astype(jnp.float32) * 2.0
    inv_freq = jnp.exp(-(i2 / DH) * math.log(THETA))
    ang = pos * inv_freq
    c = jnp.cos(ang)
    sn = jnp.sin(ang)
    cos = jnp.concatenate([c, c], axis=-1)
    sin = jnp.concatenate([sn, sn], axis=-1)

    hw = DH // 2
    for h in range(H):
        qh = q[:, h * DH:(h + 1) * DH]
        qr = jnp.concatenate([-qh[:, hw:], qh[:, :hw]], axis=-1)
        q_ref[h, :, :] = (qh * cos + qr * sin).astype(jnp.bfloat16)
        kh = k[:, h * DH:(h + 1) * DH]
        kr = jnp.concatenate([-kh[:, hw:], kh[:, :hw]], axis=-1)
        k_ref[h, :, :] = (kh * cos + kr * sin).astype(jnp.bfloat16)
        v_ref[h, :, :] = v[:, h * DH:(h + 1) * DH].astype(jnp.bfloat16)


def _qkv_call(pos, x, ln1, Wq, Wk, Wv):
    return pl.pallas_call(
        _qkv_body,
        grid=(NSB,),
        in_specs=[
            pl.BlockSpec((SB, 1), lambda i: (i, 0)),
            pl.BlockSpec((SB, D), lambda i: (i, 0)),
            pl.BlockSpec((1, D), lambda i: (0, 0)),
            pl.BlockSpec((D, H * DH), lambda i: (0, 0)),
            pl.BlockSpec((D, KVH * DH), lambda i: (0, 0)),
            pl.BlockSpec((D, KVH * DH), lambda i: (0, 0)),
        ],
        out_specs=[
            pl.BlockSpec((H, SB, DH), lambda i: (0, i, 0)),
            pl.BlockSpec((KVH, SB, DH), lambda i: (0, i, 0)),
            pl.BlockSpec((KVH, SB, DH), lambda i: (0, i, 0)),
        ],
        out_shape=[
            jax.ShapeDtypeStruct((H, S, DH), jnp.bfloat16),
            jax.ShapeDtypeStruct((KVH, S, DH), jnp.bfloat16),
            jax.ShapeDtypeStruct((KVH, S, DH), jnp.bfloat16),
        ],
        interpret=_INTERPRET,
    )(pos, x, ln1, Wq, Wk, Wv)


def _attn_body(q_ref, k_ref, v_ref, o_ref):
    qb = pl.program_id(1)
    s = jax.lax.dot_general(
        q_ref[0], k_ref[0], (((1,), (1,)), ((), ())),
        preferred_element_type=jnp.float32) * (1.0 / math.sqrt(DH))
    qpos = qb * SB + jax.lax.broadcasted_iota(jnp.int32, (SB, S), 0)
    kpos = jax.lax.broadcasted_iota(jnp.int32, (SB, S), 1)
    s = jnp.where(kpos <= qpos, s, jnp.float32(-1e9))
    m = jnp.max(s, axis=-1, keepdims=True)
    p = jnp.exp(s - m)
    p = (p / jnp.sum(p, axis=-1, keepdims=True)).astype(jnp.bfloat16)
    o_ref[0] = jnp.dot(p, v_ref[0],
                       preferred_element_type=jnp.float32).astype(jnp.bfloat16)


def _attn_call(q, k, v):
    return pl.pallas_call(
        _attn_body,
        grid=(H, NSB),
        in_specs=[
            pl.BlockSpec((1, SB, DH), lambda h, qb: (h, qb, 0)),
            pl.BlockSpec((1, S, DH), lambda h, qb: (h, 0, 0)),
            pl.BlockSpec((1, S, DH), lambda h, qb: (h, 0, 0)),
        ],
        out_specs=pl.BlockSpec((1, SB, DH), lambda h, qb: (h, qb, 0)),
        out_shape=jax.ShapeDtypeStruct((H, S, DH), jnp.bfloat16),
        interpret=_INTERPRET,
    )(q, k, v)


def _plan_body(o_ref, wo_ref, res_ref, ln_ref, wr_ref, h_ref, x2_ref,
               posA_ref, posB_ref, wA_ref, wB_ref, te_ref, tv_ref, xo_ref):
    for hh in range(H):
        xo_ref[:, hh * DH:(hh + 1) * DH] = o_ref[hh]
    x = res_ref[...] + jnp.dot(
        xo_ref[...], wo_ref[...],
        preferred_element_type=jnp.float32) * RES_SCALE
    h_ref[...] = x
    var = jnp.mean(x * x, axis=-1, keepdims=True)
    xn = ln_ref[...] * (x * jax.lax.rsqrt(var + EPS))
    x2_ref[...] = xn

    logits = jnp.dot(xn, wr_ref[...], preferred_element_type=jnp.float32)
    eio = jax.lax.broadcasted_iota(jnp.int32, (S, 128), 1)
    logits = jnp.where(eio < E, logits, jnp.float32(-1e30))
    l0 = jnp.max(logits, axis=-1, keepdims=True)
    a0 = jnp.min(jnp.where(logits == l0, eio, E), axis=-1, keepdims=True)
    lm = jnp.where(eio == a0, jnp.float32(-1e30), logits)
    l1 = jnp.max(lm, axis=-1, keepdims=True)
    a1 = jnp.min(jnp.where(lm == l1, eio, E), axis=-1, keepdims=True)
    w0 = jax.nn.sigmoid(l0 - l1)
    wA_ref[...] = w0
    wB_ref[...] = 1.0 - w0

    m0 = (eio == a0).astype(jnp.float32)
    m1 = (eio == a1).astype(jnp.float32)

    li = jax.lax.broadcasted_iota(jnp.int32, (128, 128), 0)
    lj = jax.lax.broadcasted_iota(jnp.int32, (128, 128), 1)
    tril_s = (lj < li).astype(jnp.float32)
    triu_s = (li < lj).astype(jnp.float32)

    def _col_cumsum(m, acc):
        parts = []
        for cidx in range(S // 128):
            ch = m[cidx * 128:(cidx + 1) * 128, :]
            parts.append(
                jnp.dot(tril_s, ch, preferred_element_type=jnp.float32) + acc)
            acc = acc + jnp.sum(ch, axis=0, keepdims=True)
        return jnp.concatenate(parts, axis=0), acc

    zero = jnp.zeros((1, 128), jnp.float32)
    cA, accA = _col_cumsum(m0, zero)
    cB, accB = _col_cumsum(m1, accA)
    counts = accB

    tiles = jnp.floor((counts + (M - 1)) * (1.0 / M))
    stt = jnp.dot(tiles, triu_s, preferred_element_type=jnp.float32)
    start_rows = stt * M

    posA_ref[...] = jnp.sum((start_rows + cA) * m0, axis=-1,
                            keepdims=True).astype(jnp.int32)
    posB_ref[...] = jnp.sum((start_rows + cB) * m1, axis=-1,
                            keepdims=True).astype(jnp.int32)

    jio = jax.lax.broadcasted_iota(
        jnp.int32, (128, 128), 0).astype(jnp.float32)
    sttb = jnp.broadcast_to(stt, (128, 128))
    tilb = jnp.broadcast_to(tiles, (128, 128))
    memb = jnp.logical_and(jio >= sttb, jio < sttb + tilb)
    eio_l = jax.lax.broadcasted_iota(jnp.int32, (128, 128), 1)
    te = jnp.sum(jnp.where(memb, eio_l, 0), axis=-1, keepdims=True)
    tv = (jnp.sum(memb.astype(jnp.int32), axis=-1, keepdims=True) > 0)
    te_ref[...] = jnp.where(tv, te, E - 1)
    tv_ref[...] = tv.astype(jnp.int32)


def _plan_call(o, Wo2d, res, ln2, Wr_pad):
    return pl.pallas_call(
        _plan_body,
        grid=(1,),
        in_specs=[
            pl.BlockSpec((H, S, DH), lambda i: (0, 0, 0)),
            pl.BlockSpec((H * DH, D), lambda i: (0, 0)),
            pl.BlockSpec((S, D), lambda i: (0, 0)),
            pl.BlockSpec((1, D), lambda i: (0, 0)),
            pl.BlockSpec((D, 128), lambda i: (0, 0)),
        ],
        out_specs=[
            pl.BlockSpec((S, D), lambda i: (0, 0)),
            pl.BlockSpec((S, D), lambda i: (0, 0)),
            pl.BlockSpec((S, 1), lambda i: (0, 0)),
            pl.BlockSpec((S, 1), lambda i: (0, 0)),
            pl.BlockSpec((S, 1), lambda i: (0, 0)),
            pl.BlockSpec((S, 1), lambda i: (0, 0)),
            pl.BlockSpec((128, 1), lambda i: (0, 0)),
            pl.BlockSpec((128, 1), lambda i: (0, 0)),
        ],
        out_shape=[
            jax.ShapeDtypeStruct((S, D), jnp.float32),
            jax.ShapeDtypeStruct((S, D), jnp.float32),
            jax.ShapeDtypeStruct((S, 1), jnp.int32),
            jax.ShapeDtypeStruct((S, 1), jnp.int32),
            jax.ShapeDtypeStruct((S, 1), jnp.float32),
            jax.ShapeDtypeStruct((S, 1), jnp.float32),
            jax.ShapeDtypeStruct((128, 1), jnp.int32),
            jax.ShapeDtypeStruct((128, 1), jnp.int32),
        ],
        scratch_shapes=[pltpu.VMEM((S, H * DH), jnp.bfloat16)],
        interpret=_INTERPRET,
    )(o, Wo2d, res, ln2, Wr_pad)


_NW = 32
_CHUNK = S // _NW


def _dispatch_body(x2_hbm, posA_hbm, posB_hbm, xs_hbm,
                   idxA_v, idxB_v, rows_v, semA, semB):
    wid = jax.lax.axis_index("s") * 2 + jax.lax.axis_index("c")
    base = wid * _CHUNK
    pltpu.sync_copy(posA_hbm.at[pl.ds(base, _CHUNK)], idxA_v)
    pltpu.sync_copy(posB_hbm.at[pl.ds(base, _CHUNK)], idxB_v)
    pltpu.sync_copy(x2_hbm.at[pl.ds(base, _CHUNK)], rows_v)
    cpA = pltpu.async_copy(rows_v, xs_hbm.at[idxA_v], semA)
    cpB = pltpu.async_copy(rows_v, xs_hbm.at[idxB_v], semB)
    cpA.wait()
    cpB.wait()


def _dispatch_call(x2, posA, posB):
    mesh = plsc.VectorSubcoreMesh(core_axis_name="c", subcore_axis_name="s")
    return pl.kernel(
        _dispatch_body,
        mesh=mesh,
        out_type=jax.ShapeDtypeStruct((XROWS, D), jnp.float32),
        scratch_types=[
            pltpu.VMEM((_CHUNK,), jnp.int32),
            pltpu.VMEM((_CHUNK,), jnp.int32),
            pltpu.VMEM((_CHUNK, D), jnp.float32),
            pltpu.SemaphoreType.DMA,
            pltpu.SemaphoreType.DMA,
        ],
    )(x2, posA, posB)


def _gmm_body(te_ref, tv_ref, x_ref, wg_ref, wu_ref, wd_ref, y_ref):
    i = pl.program_id(0)

    @pl.when(tv_ref[i] == 1)
    def _():
        x = x_ref[...].astype(jnp.bfloat16)
        g = jnp.dot(x, wg_ref[0], preferred_element_type=jnp.float32)
        u = jnp.dot(x, wu_ref[0], preferred_element_type=jnp.float32)
        a = (g * jax.nn.sigmoid(g) * u).astype(jnp.bfloat16)
        y_ref[...] = jnp.dot(a, wd_ref[0], preferred_element_type=jnp.float32)


def _gmm_call(te, tv, xs, Wg, Wu, Wd):
    grid_spec = pltpu.PrefetchScalarGridSpec(
        num_scalar_prefetch=2,
        grid=(NT,),
        in_specs=[
            pl.BlockSpec((M, D), lambda i, te, tv: (i, 0)),
            pl.BlockSpec((1, D, F), lambda i, te, tv: (te[i], 0, 0)),
            pl.BlockSpec((1, D, F), lambda i, te, tv: (te[i], 0, 0)),
            pl.BlockSpec((1, F, D), lambda i, te, tv: (te[i], 0, 0)),
        ],
        out_specs=pl.BlockSpec((M, D), lambda i, te, tv: (i, 0)),
    )
    return pl.pallas_call(
        _gmm_body,
        grid_spec=grid_spec,
        out_shape=jax.ShapeDtypeStruct((XROWS, D), jnp.float32),
        interpret=_INTERPRET,
    )(te, tv, xs, Wg, Wu, Wd)


def _mgather_body(ys_hbm, posA_hbm, posB_hbm, yA_hbm, yB_hbm,
                  idx_v, buf_v, sem):
    wid = jax.lax.axis_index("s") * 2 + jax.lax.axis_index("c")
    base = wid * _CHUNK
    pltpu.sync_copy(posA_hbm.at[pl.ds(base, _CHUNK)], idx_v)
    pltpu.async_copy(ys_hbm.at[idx_v], buf_v, sem).wait()
    pltpu.sync_copy(buf_v, yA_hbm.at[pl.ds(base, _CHUNK)])
    pltpu.sync_copy(posB_hbm.at[pl.ds(base, _CHUNK)], idx_v)
    pltpu.async_copy(ys_hbm.at[idx_v], buf_v, sem).wait()
    pltpu.sync_copy(buf_v, yB_hbm.at[pl.ds(base, _CHUNK)])


def _mgather_call(ys, posA, posB):
    mesh = plsc.VectorSubcoreMesh(core_axis_name="c", subcore_axis_name="s")
    return pl.kernel(
        _mgather_body,
        mesh=mesh,
        out_type=[
            jax.ShapeDtypeStruct((S, D), jnp.float32),
            jax.ShapeDtypeStruct((S, D), jnp.float32),
        ],
        scratch_types=[
            pltpu.VMEM((_CHUNK,), jnp.int32),
            pltpu.VMEM((_CHUNK, D), jnp.float32),
            pltpu.SemaphoreType.DMA,
        ],
    )(ys, posA, posB)


def _mix_body(h_ref, yA_ref, yB_ref, wA_ref, wB_ref, o_ref):
    o_ref[...] = h_ref[...] + RES_SCALE * (
        wA_ref[...] * yA_ref[...] + wB_ref[...] * yB_ref[...])


def _mix_call(h, yA, yB, wA, wB):
    return pl.pallas_call(
        _mix_body,
        grid=(NSB,),
        in_specs=[
            pl.BlockSpec((SB, D), lambda i: (i, 0)),
            pl.BlockSpec((SB, D), lambda i: (i, 0)),
            pl.BlockSpec((SB, D), lambda i: (i, 0)),
            pl.BlockSpec((SB, 1), lambda i: (i, 0)),
            pl.BlockSpec((SB, 1), lambda i: (i, 0)),
        ],
        out_specs=pl.BlockSpec((SB, D), lambda i: (i, 0)),
        out_shape=jax.ShapeDtypeStruct((S, D), jnp.float32),
        interpret=_INTERPRET,
    )(h, yA, yB, wA, wB)


@jax.jit
def _run(hidden_states, position_ids, ln1_w, ln2_w,
         Wq, Wk, Wv, Wo, Wr, Wg, Wu, Wd):
    x = hidden_states.reshape(S, D)
    pos = position_ids.reshape(S, 1)
    ln1 = ln1_w.reshape(1, D)
    ln2 = ln2_w.reshape(1, D)
    Wr_pad = jnp.pad(Wr, ((0, 0), (0, 128 - E)))

    bf = jnp.bfloat16
    q, k, v = _qkv_call(pos, x, ln1, Wq.astype(bf), Wk.astype(bf),
                        Wv.astype(bf))
    o = _attn_call(q, k, v)
    h, x2, posA, posB, wA, wB, te, tv = _plan_call(
        o, Wo.astype(bf), x, ln2, Wr_pad)
    te = te.reshape(128)
    tv = tv.reshape(128)
    posAf = posA.reshape(S)
    posBf = posB.reshape(S)
    xs = _dispatch_call(x2, posAf, posBf)
    ys = _gmm_call(te, tv, xs, Wg.astype(bf), Wu.astype(bf), Wd.astype(bf))
    yA, yB = _mgather_call(ys, posAf, posBf)
    out = _mix_call(h, yA, yB, wA, wB)
    return out.reshape(B, S, D)


def kernel(hidden_states, position_ids, ln1_w, ln2_w,
           Wq, Wk, Wv, Wo, Wr, Wg, Wu, Wd):
    return _run(hidden_states, position_ids, ln1_w, ln2_w,
                Wq, Wk, Wv, Wo, Wr, Wg, Wu, Wd)


# speedup vs baseline: 2.0526x; 1.1429x over previous
import functools
import math

import jax
import jax.numpy as jnp
from jax.experimental import pallas as pl
from jax.experimental.pallas import tpu as pltpu
from jax.experimental.pallas import tpu_sc as plsc

B, S, D = 1, 2048, 1024
H, KVH, DH = 16, 16, 64
E, K, F = 8, 2, 2048
EPS = 1e-06
THETA = 10000.0
RES_SCALE = 1.4 / math.sqrt(40.0)

SB = 256
NSB = S // SB
M = 256
NT = 24
XROWS = NT * M

_INTERPRET = False


def _qkv_body(pos_ref, x_ref, ln_ref, wq_ref, wk_ref, wv_ref,
              q_ref, k_ref, v_ref):
    x = x_ref[...]
    var = jnp.mean(x * x, axis=-1, keepdims=True)
    xn = (ln_ref[...] * (x * jax.lax.rsqrt(var + EPS))).astype(jnp.bfloat16)
    q = jnp.dot(xn, wq_ref[...], preferred_element_type=jnp.float32)
    k = jnp.dot(xn, wk_ref[...], preferred_element_type=jnp.float32)
    v = jnp.dot(xn, wv_ref[...], preferred_element_type=jnp.float32)

    pos = pos_ref[...].astype(jnp.float32)
    i2 = jax.lax.broadcasted_iota(
        jnp.int32, (1, DH // 2), 1).astype(jnp.float32) * 2.0
    inv_freq = jnp.exp(-(i2 / DH) * math.log(THETA))
    ang = pos * inv_freq
    c = jnp.cos(ang)
    sn = jnp.sin(ang)
    cos = jnp.concatenate([c, c], axis=-1)
    sin = jnp.concatenate([sn, sn], axis=-1)

    hw = DH // 2
    for h in range(H):
        qh = q[:, h * DH:(h + 1) * DH]
        qr = jnp.concatenate([-qh[:, hw:], qh[:, :hw]], axis=-1)
        q_ref[h, :, :] = (qh * cos + qr * sin).astype(jnp.bfloat16)
        kh = k[:, h * DH:(h + 1) * DH]
        kr = jnp.concatenate([-kh[:, hw:], kh[:, :hw]], axis=-1)
        k_ref[h, :, :] = (kh * cos + kr * sin).astype(jnp.bfloat16)
        v_ref[h, :, :] = v[:, h * DH:(h + 1) * DH].astype(jnp.bfloat16)


def _qkv_call(pos, x, ln1, Wq, Wk, Wv):
    return pl.pallas_call(
        _qkv_body,
        grid=(NSB,),
        in_specs=[
            pl.BlockSpec((SB, 1), lambda i: (i, 0)),
            pl.BlockSpec((SB, D), lambda i: (i, 0)),
            pl.BlockSpec((1, D), lambda i: (0, 0)),
            pl.BlockSpec((D, H * DH), lambda i: (0, 0)),
            pl.BlockSpec((D, KVH * DH), lambda i: (0, 0)),
            pl.BlockSpec((D, KVH * DH), lambda i: (0, 0)),
        ],
        out_specs=[
            pl.BlockSpec((H, SB, DH), lambda i: (0, i, 0)),
            pl.BlockSpec((KVH, SB, DH), lambda i: (0, i, 0)),
            pl.BlockSpec((KVH, SB, DH), lambda i: (0, i, 0)),
        ],
        out_shape=[
            jax.ShapeDtypeStruct((H, S, DH), jnp.bfloat16),
            jax.ShapeDtypeStruct((KVH, S, DH), jnp.bfloat16),
            jax.ShapeDtypeStruct((KVH, S, DH), jnp.bfloat16),
        ],
        interpret=_INTERPRET,
    )(pos, x, ln1, Wq, Wk, Wv)


def _attn_body(q_ref, k_ref, v_ref, o_ref):
    qb = pl.program_id(1)
    s = jax.lax.dot_general(
        q_ref[0], k_ref[0], (((1,), (1,)), ((), ())),
        preferred_element_type=jnp.float32) * (1.0 / math.sqrt(DH))
    qpos = qb * SB + jax.lax.broadcasted_iota(jnp.int32, (SB, S), 0)
    kpos = jax.lax.broadcasted_iota(jnp.int32, (SB, S), 1)
    s = jnp.where(kpos <= qpos, s, jnp.float32(-1e9))
    m = jnp.max(s, axis=-1, keepdims=True)
    p = jnp.exp(s - m)
    p = (p / jnp.sum(p, axis=-1, keepdims=True)).astype(jnp.bfloat16)
    o_ref[0] = jnp.dot(p, v_ref[0],
                       preferred_element_type=jnp.float32).astype(jnp.bfloat16)


def _attn_call(q, k, v):
    return pl.pallas_call(
        _attn_body,
        grid=(H, NSB),
        in_specs=[
            pl.BlockSpec((1, SB, DH), lambda h, qb: (h, qb, 0)),
            pl.BlockSpec((1, S, DH), lambda h, qb: (h, 0, 0)),
            pl.BlockSpec((1, S, DH), lambda h, qb: (h, 0, 0)),
        ],
        out_specs=pl.BlockSpec((1, SB, DH), lambda h, qb: (h, qb, 0)),
        out_shape=jax.ShapeDtypeStruct((H, S, DH), jnp.bfloat16),
        interpret=_INTERPRET,
    )(q, k, v)


def _plan_body(o_ref, wo_ref, res_ref, ln_ref, wr_ref, h_ref, x2_ref,
               posA_ref, posB_ref, wA_ref, wB_ref, te_ref, tv_ref, xo_ref):
    for hh in range(H):
        xo_ref[:, hh * DH:(hh + 1) * DH] = o_ref[hh]
    x = res_ref[...] + jnp.dot(
        xo_ref[...], wo_ref[...],
        preferred_element_type=jnp.float32) * RES_SCALE
    h_ref[...] = x
    var = jnp.mean(x * x, axis=-1, keepdims=True)
    xn = ln_ref[...] * (x * jax.lax.rsqrt(var + EPS))
    x2_ref[...] = xn

    logits = jnp.dot(xn, wr_ref[...], preferred_element_type=jnp.float32)
    eio = jax.lax.broadcasted_iota(jnp.int32, (S, 128), 1)
    logits = jnp.where(eio < E, logits, jnp.float32(-1e30))
    l0 = jnp.max(logits, axis=-1, keepdims=True)
    a0 = jnp.min(jnp.where(logits == l0, eio, E), axis=-1, keepdims=True)
    lm = jnp.where(eio == a0, jnp.float32(-1e30), logits)
    l1 = jnp.max(lm, axis=-1, keepdims=True)
    a1 = jnp.min(jnp.where(lm == l1, eio, E), axis=-1, keepdims=True)
    w0 = jax.nn.sigmoid(l0 - l1)
    wA_ref[...] = w0
    wB_ref[...] = 1.0 - w0

    m0 = (eio == a0).astype(jnp.float32)
    m1 = (eio == a1).astype(jnp.float32)

    li = jax.lax.broadcasted_iota(jnp.int32, (128, 128), 0)
    lj = jax.lax.broadcasted_iota(jnp.int32, (128, 128), 1)
    tril_s = (lj < li).astype(jnp.float32)
    triu_s = (li < lj).astype(jnp.float32)

    def _col_cumsum(m, acc):
        parts = []
        for cidx in range(S // 128):
            ch = m[cidx * 128:(cidx + 1) * 128, :]
            parts.append(
                jnp.dot(tril_s, ch, preferred_element_type=jnp.float32) + acc)
            acc = acc + jnp.sum(ch, axis=0, keepdims=True)
        return jnp.concatenate(parts, axis=0), acc

    zero = jnp.zeros((1, 128), jnp.float32)
    cA, accA = _col_cumsum(m0, zero)
    cB, accB = _col_cumsum(m1, accA)
    counts = accB

    tiles = jnp.floor((counts + (M - 1)) * (1.0 / M))
    stt = jnp.dot(tiles, triu_s, preferred_element_type=jnp.float32)
    start_rows = stt * M

    posA_ref[...] = jnp.sum((start_rows + cA) * m0, axis=-1,
                            keepdims=True).astype(jnp.int32)
    posB_ref[...] = jnp.sum((start_rows + cB) * m1, axis=-1,
                            keepdims=True).astype(jnp.int32)

    jio = jax.lax.broadcasted_iota(
        jnp.int32, (128, 128), 0).astype(jnp.float32)
    sttb = jnp.broadcast_to(stt, (128, 128))
    tilb = jnp.broadcast_to(tiles, (128, 128))
    memb = jnp.logical_and(jio >= sttb, jio < sttb + tilb)
    eio_l = jax.lax.broadcasted_iota(jnp.int32, (128, 128), 1)
    te = jnp.sum(jnp.where(memb, eio_l, 0), axis=-1, keepdims=True)
    tv = (jnp.sum(memb.astype(jnp.int32), axis=-1, keepdims=True) > 0)
    te_ref[...] = jnp.where(tv, te, E - 1)
    tv_ref[...] = tv.astype(jnp.int32)


def _plan_call(o, Wo2d, res, ln2, Wr_pad):
    return pl.pallas_call(
        _plan_body,
        grid=(1,),
        in_specs=[
            pl.BlockSpec((H, S, DH), lambda i: (0, 0, 0)),
            pl.BlockSpec((H * DH, D), lambda i: (0, 0)),
            pl.BlockSpec((S, D), lambda i: (0, 0)),
            pl.BlockSpec((1, D), lambda i: (0, 0)),
            pl.BlockSpec((D, 128), lambda i: (0, 0)),
        ],
        out_specs=[
            pl.BlockSpec((S, D), lambda i: (0, 0)),
            pl.BlockSpec((S, D), lambda i: (0, 0)),
            pl.BlockSpec((S, 1), lambda i: (0, 0)),
            pl.BlockSpec((S, 1), lambda i: (0, 0)),
            pl.BlockSpec((S, 1), lambda i: (0, 0)),
            pl.BlockSpec((S, 1), lambda i: (0, 0)),
            pl.BlockSpec((128, 1), lambda i: (0, 0)),
            pl.BlockSpec((128, 1), lambda i: (0, 0)),
        ],
        out_shape=[
            jax.ShapeDtypeStruct((S, D), jnp.float32),
            jax.ShapeDtypeStruct((S, D), jnp.float32),
            jax.ShapeDtypeStruct((S, 1), jnp.int32),
            jax.ShapeDtypeStruct((S, 1), jnp.int32),
            jax.ShapeDtypeStruct((S, 1), jnp.float32),
            jax.ShapeDtypeStruct((S, 1), jnp.float32),
            jax.ShapeDtypeStruct((128, 1), jnp.int32),
            jax.ShapeDtypeStruct((128, 1), jnp.int32),
        ],
        scratch_shapes=[pltpu.VMEM((S, H * DH), jnp.bfloat16)],
        interpret=_INTERPRET,
    )(o, Wo2d, res, ln2, Wr_pad)


_NW = 32
_CHUNK = S // _NW


def _dispatch_body(x2_hbm, posA_hbm, posB_hbm, xs_hbm,
                   idxA_v, idxB_v, rows_v, semA, semB):
    wid = jax.lax.axis_index("s") * 2 + jax.lax.axis_index("c")
    base = wid * _CHUNK
    pltpu.sync_copy(posA_hbm.at[pl.ds(base, _CHUNK)], idxA_v)
    pltpu.sync_copy(posB_hbm.at[pl.ds(base, _CHUNK)], idxB_v)
    pltpu.sync_copy(x2_hbm.at[pl.ds(base, _CHUNK)], rows_v)
    cpA = pltpu.async_copy(rows_v, xs_hbm.at[idxA_v], semA)
    cpB = pltpu.async_copy(rows_v, xs_hbm.at[idxB_v], semB)
    cpA.wait()
    cpB.wait()


def _dispatch_call(x2, posA, posB):
    mesh = plsc.VectorSubcoreMesh(core_axis_name="c", subcore_axis_name="s")
    return pl.kernel(
        _dispatch_body,
        mesh=mesh,
        out_type=jax.ShapeDtypeStruct((XROWS, D), jnp.float32),
        scratch_types=[
            pltpu.VMEM((_CHUNK,), jnp.int32),
            pltpu.VMEM((_CHUNK,), jnp.int32),
            pltpu.VMEM((_CHUNK, D), jnp.float32),
            pltpu.SemaphoreType.DMA,
            pltpu.SemaphoreType.DMA,
        ],
    )(x2, posA, posB)


def _gmm_body(te_ref, tv_ref, x_ref, wg_ref, wu_ref, wd_ref, y_ref):
    i = pl.program_id(0)

    @pl.when(tv_ref[i] == 1)
    def _():
        x = x_ref[...]
        g = jnp.dot(x, wg_ref[0], preferred_element_type=jnp.float32)
        u = jnp.dot(x, wu_ref[0], preferred_element_type=jnp.float32)
        a = g * jax.nn.sigmoid(g) * u
        y_ref[...] = jnp.dot(a, wd_ref[0], preferred_element_type=jnp.float32)


def _gmm_call(te, tv, xs, Wg, Wu, Wd):
    grid_spec = pltpu.PrefetchScalarGridSpec(
        num_scalar_prefetch=2,
        grid=(NT,),
        in_specs=[
            pl.BlockSpec((M, D), lambda i, te, tv: (i, 0)),
            pl.BlockSpec((1, D, F), lambda i, te, tv: (te[i], 0, 0)),
            pl.BlockSpec((1, D, F), lambda i, te, tv: (te[i], 0, 0)),
            pl.BlockSpec((1, F, D), lambda i, te, tv: (te[i], 0, 0)),
        ],
        out_specs=pl.BlockSpec((M, D), lambda i, te, tv: (i, 0)),
    )
    return pl.pallas_call(
        _gmm_body,
        grid_spec=grid_spec,
        out_shape=jax.ShapeDtypeStruct((XROWS, D), jnp.float32),
        interpret=_INTERPRET,
    )(te, tv, xs, Wg, Wu, Wd)


def _mgather_body(ys_hbm, posA_hbm, posB_hbm, yA_hbm, yB_hbm,
                  idx_v, buf_v, sem):
    wid = jax.lax.axis_index("s") * 2 + jax.lax.axis_index("c")
    base = wid * _CHUNK
    pltpu.sync_copy(posA_hbm.at[pl.ds(base, _CHUNK)], idx_v)
    pltpu.async_copy(ys_hbm.at[idx_v], buf_v, sem).wait()
    pltpu.sync_copy(buf_v, yA_hbm.at[pl.ds(base, _CHUNK)])
    pltpu.sync_copy(posB_hbm.at[pl.ds(base, _CHUNK)], idx_v)
    pltpu.async_copy(ys_hbm.at[idx_v], buf_v, sem).wait()
    pltpu.sync_copy(buf_v, yB_hbm.at[pl.ds(base, _CHUNK)])


def _mgather_call(ys, posA, posB):
    mesh = plsc.VectorSubcoreMesh(core_axis_name="c", subcore_axis_name="s")
    return pl.kernel(
        _mgather_body,
        mesh=mesh,
        out_type=[
            jax.ShapeDtypeStruct((S, D), jnp.float32),
            jax.ShapeDtypeStruct((S, D), jnp.float32),
        ],
        scratch_types=[
            pltpu.VMEM((_CHUNK,), jnp.int32),
            pltpu.VMEM((_CHUNK, D), jnp.float32),
            pltpu.SemaphoreType.DMA,
        ],
    )(ys, posA, posB)


def _mix_body(h_ref, yA_ref, yB_ref, wA_ref, wB_ref, o_ref):
    o_ref[...] = h_ref[...] + RES_SCALE * (
        wA_ref[...] * yA_ref[...] + wB_ref[...] * yB_ref[...])


def _mix_call(h, yA, yB, wA, wB):
    return pl.pallas_call(
        _mix_body,
        grid=(NSB,),
        in_specs=[
            pl.BlockSpec((SB, D), lambda i: (i, 0)),
            pl.BlockSpec((SB, D), lambda i: (i, 0)),
            pl.BlockSpec((SB, D), lambda i: (i, 0)),
            pl.BlockSpec((SB, 1), lambda i: (i, 0)),
            pl.BlockSpec((SB, 1), lambda i: (i, 0)),
        ],
        out_specs=pl.BlockSpec((SB, D), lambda i: (i, 0)),
        out_shape=jax.ShapeDtypeStruct((S, D), jnp.float32),
        interpret=_INTERPRET,
    )(h, yA, yB, wA, wB)


@jax.jit
def _run(hidden_states, position_ids, ln1_w, ln2_w,
         Wq, Wk, Wv, Wo, Wr, Wg, Wu, Wd):
    x = hidden_states.reshape(S, D)
    pos = position_ids.reshape(S, 1)
    ln1 = ln1_w.reshape(1, D)
    ln2 = ln2_w.reshape(1, D)
    Wr_pad = jnp.pad(Wr, ((0, 0), (0, 128 - E)))

    bf = jnp.bfloat16
    q, k, v = _qkv_call(pos, x, ln1, Wq.astype(bf), Wk.astype(bf),
                        Wv.astype(bf))
    o = _attn_call(q, k, v)
    h, x2, posA, posB, wA, wB, te, tv = _plan_call(
        o, Wo.astype(bf), x, ln2, Wr_pad)
    te = te.reshape(128)
    tv = tv.reshape(128)
    posAf = posA.reshape(S)
    posBf = posB.reshape(S)
    xs = _dispatch_call(x2, posAf, posBf)
    ys = _gmm_call(te, tv, xs, Wg, Wu, Wd)
    yA, yB = _mgather_call(ys, posAf, posBf)
    out = _mix_call(h, yA, yB, wA, wB)
    return out.reshape(B, S, D)


def kernel(hidden_states, position_ids, ln1_w, ln2_w,
           Wq, Wk, Wv, Wo, Wr, Wg, Wu, Wd):
    return _run(hidden_states, position_ids, ln1_w, ln2_w,
                Wq, Wk, Wv, Wo, Wr, Wg, Wu, Wd)


# speedup vs baseline: 2.1800x; 1.0620x over previous
import functools
import math

import jax
import jax.numpy as jnp
from jax.experimental import pallas as pl
from jax.experimental.pallas import tpu as pltpu
from jax.experimental.pallas import tpu_sc as plsc

B, S, D = 1, 2048, 1024
H, KVH, DH = 16, 16, 64
E, K, F = 8, 2, 2048
EPS = 1e-06
THETA = 10000.0
RES_SCALE = 1.4 / math.sqrt(40.0)

SB = 256
NSB = S // SB
M = 256
NT = 24
XROWS = NT * M

_INTERPRET = False


def _qkv_body(pos_ref, x_ref, ln_ref, wq_ref, wk_ref, wv_ref,
              q_ref, k_ref, v_ref):
    x = x_ref[...]
    var = jnp.mean(x * x, axis=-1, keepdims=True)
    xn = (ln_ref[...] * (x * jax.lax.rsqrt(var + EPS))).astype(jnp.bfloat16)
    q = jnp.dot(xn, wq_ref[...], preferred_element_type=jnp.float32)
    k = jnp.dot(xn, wk_ref[...], preferred_element_type=jnp.float32)
    v = jnp.dot(xn, wv_ref[...], preferred_element_type=jnp.float32)

    pos = pos_ref[...].astype(jnp.float32)
    i2 = jax.lax.broadcasted_iota(
        jnp.int32, (1, DH // 2), 1).astype(jnp.float32) * 2.0
    inv_freq = jnp.exp(-(i2 / DH) * math.log(THETA))
    ang = pos * inv_freq
    c = jnp.cos(ang)
    sn = jnp.sin(ang)
    cos = jnp.concatenate([c, c], axis=-1)
    sin = jnp.concatenate([sn, sn], axis=-1)

    hw = DH // 2
    for h in range(H):
        qh = q[:, h * DH:(h + 1) * DH]
        qr = jnp.concatenate([-qh[:, hw:], qh[:, :hw]], axis=-1)
        q_ref[h, :, :] = (qh * cos + qr * sin).astype(jnp.bfloat16)
        kh = k[:, h * DH:(h + 1) * DH]
        kr = jnp.concatenate([-kh[:, hw:], kh[:, :hw]], axis=-1)
        k_ref[h, :, :] = (kh * cos + kr * sin).astype(jnp.bfloat16)
        v_ref[h, :, :] = v[:, h * DH:(h + 1) * DH].astype(jnp.bfloat16)


def _qkv_call(pos, x, ln1, Wq, Wk, Wv):
    return pl.pallas_call(
        _qkv_body,
        grid=(NSB,),
        in_specs=[
            pl.BlockSpec((SB, 1), lambda i: (i, 0)),
            pl.BlockSpec((SB, D), lambda i: (i, 0)),
            pl.BlockSpec((1, D), lambda i: (0, 0)),
            pl.BlockSpec((D, H * DH), lambda i: (0, 0)),
            pl.BlockSpec((D, KVH * DH), lambda i: (0, 0)),
            pl.BlockSpec((D, KVH * DH), lambda i: (0, 0)),
        ],
        out_specs=[
            pl.BlockSpec((H, SB, DH), lambda i: (0, i, 0)),
            pl.BlockSpec((KVH, SB, DH), lambda i: (0, i, 0)),
            pl.BlockSpec((KVH, SB, DH), lambda i: (0, i, 0)),
        ],
        out_shape=[
            jax.ShapeDtypeStruct((H, S, DH), jnp.bfloat16),
            jax.ShapeDtypeStruct((KVH, S, DH), jnp.bfloat16),
            jax.ShapeDtypeStruct((KVH, S, DH), jnp.bfloat16),
        ],
        interpret=_INTERPRET,
    )(pos, x, ln1, Wq, Wk, Wv)


def _make_attn_body(qb_off, klen):
    def body(q_ref, k_ref, v_ref, o_ref):
        qb = pl.program_id(1) + qb_off
        s = jax.lax.dot_general(
            q_ref[0], k_ref[0], (((1,), (1,)), ((), ())),
            preferred_element_type=jnp.float32) * (1.0 / math.sqrt(DH))
        qpos = qb * SB + jax.lax.broadcasted_iota(jnp.int32, (SB, klen), 0)
        kpos = jax.lax.broadcasted_iota(jnp.int32, (SB, klen), 1)
        s = jnp.where(kpos <= qpos, s, jnp.float32(-1e9))
        m = jnp.max(s, axis=-1, keepdims=True)
        p = jnp.exp(s - m)
        p = (p / jnp.sum(p, axis=-1, keepdims=True)).astype(jnp.bfloat16)
        o_ref[0] = jnp.dot(
            p, v_ref[0],
            preferred_element_type=jnp.float32).astype(jnp.bfloat16)
    return body


def _attn_call(q, k, v):
    nh = NSB // 2
    oA = pl.pallas_call(
        _make_attn_body(0, S // 2),
        grid=(H, nh),
        in_specs=[
            pl.BlockSpec((1, SB, DH), lambda h, qb: (h, qb, 0)),
            pl.BlockSpec((1, S // 2, DH), lambda h, qb: (h, 0, 0)),
            pl.BlockSpec((1, S // 2, DH), lambda h, qb: (h, 0, 0)),
        ],
        out_specs=pl.BlockSpec((1, SB, DH), lambda h, qb: (h, qb, 0)),
        out_shape=jax.ShapeDtypeStruct((H, S // 2, DH), jnp.bfloat16),
        interpret=_INTERPRET,
    )(q, k, v)
    oB = pl.pallas_call(
        _make_attn_body(nh, S),
        grid=(H, nh),
        in_specs=[
            pl.BlockSpec((1, SB, DH), lambda h, qb: (h, qb + NSB // 2, 0)),
            pl.BlockSpec((1, S, DH), lambda h, qb: (h, 0, 0)),
            pl.BlockSpec((1, S, DH), lambda h, qb: (h, 0, 0)),
        ],
        out_specs=pl.BlockSpec((1, SB, DH), lambda h, qb: (h, qb, 0)),
        out_shape=jax.ShapeDtypeStruct((H, S // 2, DH), jnp.bfloat16),
        interpret=_INTERPRET,
    )(q, k, v)
    return oA, oB


def _plan_body(oA_ref, oB_ref, wo_ref, res_ref, ln_ref, wr_ref, h_ref,
               x2_ref, posA_ref, posB_ref, wA_ref, wB_ref, te_ref, tv_ref,
               xo_ref):
    for hh in range(H):
        xo_ref[:S // 2, hh * DH:(hh + 1) * DH] = oA_ref[hh]
        xo_ref[S // 2:, hh * DH:(hh + 1) * DH] = oB_ref[hh]
    x = res_ref[...] + jnp.dot(
        xo_ref[...], wo_ref[...],
        preferred_element_type=jnp.float32) * RES_SCALE
    h_ref[...] = x
    var = jnp.mean(x * x, axis=-1, keepdims=True)
    xn = ln_ref[...] * (x * jax.lax.rsqrt(var + EPS))
    x2_ref[...] = xn

    logits = jnp.dot(xn, wr_ref[...], preferred_element_type=jnp.float32)
    eio = jax.lax.broadcasted_iota(jnp.int32, (S, 128), 1)
    logits = jnp.where(eio < E, logits, jnp.float32(-1e30))
    l0 = jnp.max(logits, axis=-1, keepdims=True)
    a0 = jnp.min(jnp.where(logits == l0, eio, E), axis=-1, keepdims=True)
    lm = jnp.where(eio == a0, jnp.float32(-1e30), logits)
    l1 = jnp.max(lm, axis=-1, keepdims=True)
    a1 = jnp.min(jnp.where(lm == l1, eio, E), axis=-1, keepdims=True)
    w0 = jax.nn.sigmoid(l0 - l1)
    wA_ref[...] = w0
    wB_ref[...] = 1.0 - w0

    m0 = (eio == a0).astype(jnp.float32)
    m1 = (eio == a1).astype(jnp.float32)

    li = jax.lax.broadcasted_iota(jnp.int32, (128, 128), 0)
    lj = jax.lax.broadcasted_iota(jnp.int32, (128, 128), 1)
    tril_s = (lj < li).astype(jnp.float32)
    triu_s = (li < lj).astype(jnp.float32)

    def _col_cumsum(m, acc):
        parts = []
        for cidx in range(S // 128):
            ch = m[cidx * 128:(cidx + 1) * 128, :]
            parts.append(
                jnp.dot(tril_s, ch, preferred_element_type=jnp.float32) + acc)
            acc = acc + jnp.sum(ch, axis=0, keepdims=True)
        return jnp.concatenate(parts, axis=0), acc

    zero = jnp.zeros((1, 128), jnp.float32)
    cA, accA = _col_cumsum(m0, zero)
    cB, accB = _col_cumsum(m1, accA)
    counts = accB

    tiles = jnp.floor((counts + (M - 1)) * (1.0 / M))
    stt = jnp.dot(tiles, triu_s, preferred_element_type=jnp.float32)
    start_rows = stt * M

    posA_ref[...] = jnp.sum((start_rows + cA) * m0, axis=-1,
                            keepdims=True).astype(jnp.int32)
    posB_ref[...] = jnp.sum((start_rows + cB) * m1, axis=-1,
                            keepdims=True).astype(jnp.int32)

    jio = jax.lax.broadcasted_iota(
        jnp.int32, (128, 128), 0).astype(jnp.float32)
    sttb = jnp.broadcast_to(stt, (128, 128))
    tilb = jnp.broadcast_to(tiles, (128, 128))
    memb = jnp.logical_and(jio >= sttb, jio < sttb + tilb)
    eio_l = jax.lax.broadcasted_iota(jnp.int32, (128, 128), 1)
    te = jnp.sum(jnp.where(memb, eio_l, 0), axis=-1, keepdims=True)
    tv = (jnp.sum(memb.astype(jnp.int32), axis=-1, keepdims=True) > 0)
    te_ref[...] = jnp.where(tv, te, E - 1)
    tv_ref[...] = tv.astype(jnp.int32)


def _plan_call(oA, oB, Wo2d, res, ln2, Wr_pad):
    return pl.pallas_call(
        _plan_body,
        grid=(1,),
        in_specs=[
            pl.BlockSpec((H, S // 2, DH), lambda i: (0, 0, 0)),
            pl.BlockSpec((H, S // 2, DH), lambda i: (0, 0, 0)),
            pl.BlockSpec((H * DH, D), lambda i: (0, 0)),
            pl.BlockSpec((S, D), lambda i: (0, 0)),
            pl.BlockSpec((1, D), lambda i: (0, 0)),
            pl.BlockSpec((D, 128), lambda i: (0, 0)),
        ],
        out_specs=[
            pl.BlockSpec((S, D), lambda i: (0, 0)),
            pl.BlockSpec((S, D), lambda i: (0, 0)),
            pl.BlockSpec((S, 1), lambda i: (0, 0)),
            pl.BlockSpec((S, 1), lambda i: (0, 0)),
            pl.BlockSpec((S, 1), lambda i: (0, 0)),
            pl.BlockSpec((S, 1), lambda i: (0, 0)),
            pl.BlockSpec((128, 1), lambda i: (0, 0)),
            pl.BlockSpec((128, 1), lambda i: (0, 0)),
        ],
        out_shape=[
            jax.ShapeDtypeStruct((S, D), jnp.float32),
            jax.ShapeDtypeStruct((S, D), jnp.float32),
            jax.ShapeDtypeStruct((S, 1), jnp.int32),
            jax.ShapeDtypeStruct((S, 1), jnp.int32),
            jax.ShapeDtypeStruct((S, 1), jnp.float32),
            jax.ShapeDtypeStruct((S, 1), jnp.float32),
            jax.ShapeDtypeStruct((128, 1), jnp.int32),
            jax.ShapeDtypeStruct((128, 1), jnp.int32),
        ],
        scratch_shapes=[pltpu.VMEM((S, H * DH), jnp.bfloat16)],
        interpret=_INTERPRET,
    )(oA, oB, Wo2d, res, ln2, Wr_pad)


_NW = 32
_CHUNK = S // _NW


def _dispatch_body(x2_hbm, posA_hbm, posB_hbm, xs_hbm,
                   idxA_v, idxB_v, rows_v, semA, semB):
    wid = jax.lax.axis_index("s") * 2 + jax.lax.axis_index("c")
    base = wid * _CHUNK
    pltpu.sync_copy(posA_hbm.at[pl.ds(base, _CHUNK)], idxA_v)
    pltpu.sync_copy(posB_hbm.at[pl.ds(base, _CHUNK)], idxB_v)
    pltpu.sync_copy(x2_hbm.at[pl.ds(base, _CHUNK)], rows_v)
    cpA = pltpu.async_copy(rows_v, xs_hbm.at[idxA_v], semA)
    cpB = pltpu.async_copy(rows_v, xs_hbm.at[idxB_v], semB)
    cpA.wait()
    cpB.wait()


def _dispatch_call(x2, posA, posB):
    mesh = plsc.VectorSubcoreMesh(core_axis_name="c", subcore_axis_name="s")
    return pl.kernel(
        _dispatch_body,
        mesh=mesh,
        out_type=jax.ShapeDtypeStruct((XROWS, D), jnp.float32),
        scratch_types=[
            pltpu.VMEM((_CHUNK,), jnp.int32),
            pltpu.VMEM((_CHUNK,), jnp.int32),
            pltpu.VMEM((_CHUNK, D), jnp.float32),
            pltpu.SemaphoreType.DMA,
            pltpu.SemaphoreType.DMA,
        ],
    )(x2, posA, posB)


def _gmm_body(te_ref, tv_ref, x_ref, wg_ref, wu_ref, wd_ref, y_ref):
    i = pl.program_id(0)

    @pl.when(tv_ref[i] == 1)
    def _():
        x = x_ref[...]
        g = jnp.dot(x, wg_ref[0], preferred_element_type=jnp.float32)
        u = jnp.dot(x, wu_ref[0], preferred_element_type=jnp.float32)
        a = g * jax.nn.sigmoid(g) * u
        y_ref[...] = jnp.dot(a, wd_ref[0], preferred_element_type=jnp.float32)


def _gmm_call(te, tv, xs, Wg, Wu, Wd):
    grid_spec = pltpu.PrefetchScalarGridSpec(
        num_scalar_prefetch=2,
        grid=(NT,),
        in_specs=[
            pl.BlockSpec((M, D), lambda i, te, tv: (i, 0)),
            pl.BlockSpec((1, D, F), lambda i, te, tv: (te[i], 0, 0)),
            pl.BlockSpec((1, D, F), lambda i, te, tv: (te[i], 0, 0)),
            pl.BlockSpec((1, F, D), lambda i, te, tv: (te[i], 0, 0)),
        ],
        out_specs=pl.BlockSpec((M, D), lambda i, te, tv: (i, 0)),
    )
    return pl.pallas_call(
        _gmm_body,
        grid_spec=grid_spec,
        out_shape=jax.ShapeDtypeStruct((XROWS, D), jnp.float32),
        interpret=_INTERPRET,
    )(te, tv, xs, Wg, Wu, Wd)


def _mgather_body(ys_hbm, posA_hbm, posB_hbm, yA_hbm, yB_hbm,
                  idx_v, buf_v, sem):
    wid = jax.lax.axis_index("s") * 2 + jax.lax.axis_index("c")
    base = wid * _CHUNK
    pltpu.sync_copy(posA_hbm.at[pl.ds(base, _CHUNK)], idx_v)
    pltpu.async_copy(ys_hbm.at[idx_v], buf_v, sem).wait()
    pltpu.sync_copy(buf_v, yA_hbm.at[pl.ds(base, _CHUNK)])
    pltpu.sync_copy(posB_hbm.at[pl.ds(base, _CHUNK)], idx_v)
    pltpu.async_copy(ys_hbm.at[idx_v], buf_v, sem).wait()
    pltpu.sync_copy(buf_v, yB_hbm.at[pl.ds(base, _CHUNK)])


def _mgather_call(ys, posA, posB):
    mesh = plsc.VectorSubcoreMesh(core_axis_name="c", subcore_axis_name="s")
    return pl.kernel(
        _mgather_body,
        mesh=mesh,
        out_type=[
            jax.ShapeDtypeStruct((S, D), jnp.float32),
            jax.ShapeDtypeStruct((S, D), jnp.float32),
        ],
        scratch_types=[
            pltpu.VMEM((_CHUNK,), jnp.int32),
            pltpu.VMEM((_CHUNK, D), jnp.float32),
            pltpu.SemaphoreType.DMA,
        ],
    )(ys, posA, posB)


def _mix_body(h_ref, yA_ref, yB_ref, wA_ref, wB_ref, o_ref):
    o_ref[...] = h_ref[...] + RES_SCALE * (
        wA_ref[...] * yA_ref[...] + wB_ref[...] * yB_ref[...])


def _mix_call(h, yA, yB, wA, wB):
    return pl.pallas_call(
        _mix_body,
        grid=(NSB,),
        in_specs=[
            pl.BlockSpec((SB, D), lambda i: (i, 0)),
            pl.BlockSpec((SB, D), lambda i: (i, 0)),
            pl.BlockSpec((SB, D), lambda i: (i, 0)),
            pl.BlockSpec((SB, 1), lambda i: (i, 0)),
            pl.BlockSpec((SB, 1), lambda i: (i, 0)),
        ],
        out_specs=pl.BlockSpec((SB, D), lambda i: (i, 0)),
        out_shape=jax.ShapeDtypeStruct((S, D), jnp.float32),
        interpret=_INTERPRET,
    )(h, yA, yB, wA, wB)


@jax.jit
def _run(hidden_states, position_ids, ln1_w, ln2_w,
         Wq, Wk, Wv, Wo, Wr, Wg, Wu, Wd):
    x = hidden_states.reshape(S, D)
    pos = position_ids.reshape(S, 1)
    ln1 = ln1_w.reshape(1, D)
    ln2 = ln2_w.reshape(1, D)
    Wr_pad = jnp.pad(Wr, ((0, 0), (0, 128 - E)))

    bf = jnp.bfloat16
    q, k, v = _qkv_call(pos, x, ln1, Wq.astype(bf), Wk.astype(bf),
                        Wv.astype(bf))
    oA, oB = _attn_call(q, k, v)
    h, x2, posA, posB, wA, wB, te, tv = _plan_call(
        oA, oB, Wo.astype(bf), x, ln2, Wr_pad)
    te = te.reshape(128)
    tv = tv.reshape(128)
    posAf = posA.reshape(S)
    posBf = posB.reshape(S)
    xs = _dispatch_call(x2, posAf, posBf)
    ys = _gmm_call(te, tv, xs, Wg, Wu, Wd)
    yA, yB = _mgather_call(ys, posAf, posBf)
    out = _mix_call(h, yA, yB, wA, wB)
    return out.reshape(B, S, D)


def kernel(hidden_states, position_ids, ln1_w, ln2_w,
           Wq, Wk, Wv, Wo, Wr, Wg, Wu, Wd):
    return _run(hidden_states, position_ids, ln1_w, ln2_w,
                Wq, Wk, Wv, Wo, Wr, Wg, Wu, Wd)


# speedup vs baseline: 2.1830x; 1.0014x over previous
import functools
import math

import jax
import jax.numpy as jnp
from jax.experimental import pallas as pl
from jax.experimental.pallas import tpu as pltpu
from jax.experimental.pallas import tpu_sc as plsc

B, S, D = 1, 2048, 1024
H, KVH, DH = 16, 16, 64
E, K, F = 8, 2, 2048
EPS = 1e-06
THETA = 10000.0
RES_SCALE = 1.4 / math.sqrt(40.0)

SB = 256
NSB = S // SB
M = 256
NT = 24
XROWS = NT * M

_INTERPRET = False


def _qkv_body(pos_ref, x_ref, ln_ref, wq_ref, wk_ref, wv_ref,
              q_ref, k_ref, v_ref):
    x = x_ref[...]
    var = jnp.mean(x * x, axis=-1, keepdims=True)
    xn = (ln_ref[...] * (x * jax.lax.rsqrt(var + EPS))).astype(jnp.bfloat16)
    q = jnp.dot(xn, wq_ref[...], preferred_element_type=jnp.float32)
    k = jnp.dot(xn, wk_ref[...], preferred_element_type=jnp.float32)
    v = jnp.dot(xn, wv_ref[...], preferred_element_type=jnp.float32)

    pos = pos_ref[...].astype(jnp.float32)
    i2 = jax.lax.broadcasted_iota(
        jnp.int32, (1, DH // 2), 1).astype(jnp.float32) * 2.0
    inv_freq = jnp.exp(-(i2 / DH) * math.log(THETA))
    ang = pos * inv_freq
    c = jnp.cos(ang)
    sn = jnp.sin(ang)
    cos = jnp.concatenate([c, c], axis=-1)
    sin = jnp.concatenate([sn, sn], axis=-1)

    hw = DH // 2
    for h in range(H):
        qh = q[:, h * DH:(h + 1) * DH]
        qr = jnp.concatenate([-qh[:, hw:], qh[:, :hw]], axis=-1)
        q_ref[h, :, :] = (qh * cos + qr * sin).astype(jnp.bfloat16)
        kh = k[:, h * DH:(h + 1) * DH]
        kr = jnp.concatenate([-kh[:, hw:], kh[:, :hw]], axis=-1)
        k_ref[h, :, :] = (kh * cos + kr * sin).astype(jnp.bfloat16)
        v_ref[h, :, :] = v[:, h * DH:(h + 1) * DH].astype(jnp.bfloat16)


def _qkv_call(pos, x, ln1, Wq, Wk, Wv):
    return pl.pallas_call(
        _qkv_body,
        grid=(NSB,),
        in_specs=[
            pl.BlockSpec((SB, 1), lambda i: (i, 0)),
            pl.BlockSpec((SB, D), lambda i: (i, 0)),
            pl.BlockSpec((1, D), lambda i: (0, 0)),
            pl.BlockSpec((D, H * DH), lambda i: (0, 0)),
            pl.BlockSpec((D, KVH * DH), lambda i: (0, 0)),
            pl.BlockSpec((D, KVH * DH), lambda i: (0, 0)),
        ],
        out_specs=[
            pl.BlockSpec((H, SB, DH), lambda i: (0, i, 0)),
            pl.BlockSpec((KVH, SB, DH), lambda i: (0, i, 0)),
            pl.BlockSpec((KVH, SB, DH), lambda i: (0, i, 0)),
        ],
        out_shape=[
            jax.ShapeDtypeStruct((H, S, DH), jnp.bfloat16),
            jax.ShapeDtypeStruct((KVH, S, DH), jnp.bfloat16),
            jax.ShapeDtypeStruct((KVH, S, DH), jnp.bfloat16),
        ],
        interpret=_INTERPRET,
    )(pos, x, ln1, Wq, Wk, Wv)


def _make_attn_body(qb_off, klen):
    def body(q_ref, k_ref, v_ref, o_ref):
        qb = pl.program_id(1) + qb_off
        s = jax.lax.dot_general(
            q_ref[0], k_ref[0], (((1,), (1,)), ((), ())),
            preferred_element_type=jnp.float32) * (1.0 / math.sqrt(DH))
        qpos = qb * SB + jax.lax.broadcasted_iota(jnp.int32, (SB, klen), 0)
        kpos = jax.lax.broadcasted_iota(jnp.int32, (SB, klen), 1)
        s = jnp.where(kpos <= qpos, s, jnp.float32(-1e9))
        m = jnp.max(s, axis=-1, keepdims=True)
        p = jnp.exp(s - m)
        p = (p / jnp.sum(p, axis=-1, keepdims=True)).astype(jnp.bfloat16)
        o_ref[0] = jnp.dot(
            p, v_ref[0],
            preferred_element_type=jnp.float32).astype(jnp.bfloat16)
    return body


def _attn_call(q, k, v):
    nh = NSB // 2
    oA = pl.pallas_call(
        _make_attn_body(0, S // 2),
        grid=(H, nh),
        in_specs=[
            pl.BlockSpec((1, SB, DH), lambda h, qb: (h, qb, 0)),
            pl.BlockSpec((1, S // 2, DH), lambda h, qb: (h, 0, 0)),
            pl.BlockSpec((1, S // 2, DH), lambda h, qb: (h, 0, 0)),
        ],
        out_specs=pl.BlockSpec((1, SB, DH), lambda h, qb: (h, qb, 0)),
        out_shape=jax.ShapeDtypeStruct((H, S // 2, DH), jnp.bfloat16),
        interpret=_INTERPRET,
    )(q, k, v)
    oB = pl.pallas_call(
        _make_attn_body(nh, S),
        grid=(H, nh),
        in_specs=[
            pl.BlockSpec((1, SB, DH), lambda h, qb: (h, qb + NSB // 2, 0)),
            pl.BlockSpec((1, S, DH), lambda h, qb: (h, 0, 0)),
            pl.BlockSpec((1, S, DH), lambda h, qb: (h, 0, 0)),
        ],
        out_specs=pl.BlockSpec((1, SB, DH), lambda h, qb: (h, qb, 0)),
        out_shape=jax.ShapeDtypeStruct((H, S // 2, DH), jnp.bfloat16),
        interpret=_INTERPRET,
    )(q, k, v)
    return oA, oB


def _plan_body(oA_ref, oB_ref, wo_ref, res_ref, ln_ref, wr_ref, h_ref,
               x2_ref, posA_ref, posB_ref, wA_ref, wB_ref, te_ref, tv_ref,
               xo_ref):
    for hh in range(H):
        xo_ref[:S // 2, hh * DH:(hh + 1) * DH] = oA_ref[hh]
        xo_ref[S // 2:, hh * DH:(hh + 1) * DH] = oB_ref[hh]
    x = res_ref[...] + jnp.dot(
        xo_ref[...], wo_ref[...],
        preferred_element_type=jnp.float32) * RES_SCALE
    h_ref[...] = x
    var = jnp.mean(x * x, axis=-1, keepdims=True)
    xn = ln_ref[...] * (x * jax.lax.rsqrt(var + EPS))
    x2_ref[...] = xn

    logits = jnp.dot(xn, wr_ref[...], preferred_element_type=jnp.float32)
    eio = jax.lax.broadcasted_iota(jnp.int32, (S, 128), 1)
    logits = jnp.where(eio < E, logits, jnp.float32(-1e30))
    l0 = jnp.max(logits, axis=-1, keepdims=True)
    a0 = jnp.min(jnp.where(logits == l0, eio, E), axis=-1, keepdims=True)
    lm = jnp.where(eio == a0, jnp.float32(-1e30), logits)
    l1 = jnp.max(lm, axis=-1, keepdims=True)
    a1 = jnp.min(jnp.where(lm == l1, eio, E), axis=-1, keepdims=True)
    w0 = jax.nn.sigmoid(l0 - l1)
    wA_ref[...] = w0
    wB_ref[...] = 1.0 - w0

    m0 = (eio == a0).astype(jnp.float32)
    m1 = (eio == a1).astype(jnp.float32)

    li = jax.lax.broadcasted_iota(jnp.int32, (128, 128), 0)
    lj = jax.lax.broadcasted_iota(jnp.int32, (128, 128), 1)
    tril_s = (lj < li).astype(jnp.float32)
    triu_s = (li < lj).astype(jnp.float32)

    def _col_cumsum(m, acc):
        parts = []
        for cidx in range(S // 128):
            ch = m[cidx * 128:(cidx + 1) * 128, :]
            parts.append(
                jnp.dot(tril_s, ch, preferred_element_type=jnp.float32) + acc)
            acc = acc + jnp.sum(ch, axis=0, keepdims=True)
        return jnp.concatenate(parts, axis=0), acc

    zero = jnp.zeros((1, 128), jnp.float32)
    cA, accA = _col_cumsum(m0, zero)
    cB, accB = _col_cumsum(m1, accA)
    counts = accB

    tiles = jnp.floor((counts + (M - 1)) * (1.0 / M))
    stt = jnp.dot(tiles, triu_s, preferred_element_type=jnp.float32)
    start_rows = stt * M

    posA_ref[...] = jnp.sum((start_rows + cA) * m0, axis=-1,
                            keepdims=True).astype(jnp.int32)
    posB_ref[...] = jnp.sum((start_rows + cB) * m1, axis=-1,
                            keepdims=True).astype(jnp.int32)

    jio = jax.lax.broadcasted_iota(
        jnp.int32, (128, 128), 0).astype(jnp.float32)
    sttb = jnp.broadcast_to(stt, (128, 128))
    tilb = jnp.broadcast_to(tiles, (128, 128))
    memb = jnp.logical_and(jio >= sttb, jio < sttb + tilb)
    eio_l = jax.lax.broadcasted_iota(jnp.int32, (128, 128), 1)
    te = jnp.sum(jnp.where(memb, eio_l, 0), axis=-1, keepdims=True)
    tv = (jnp.sum(memb.astype(jnp.int32), axis=-1, keepdims=True) > 0)
    te_ref[...] = jnp.where(tv, te, E - 1)
    tv_ref[...] = tv.astype(jnp.int32)


def _plan_call(oA, oB, Wo2d, res, ln2, Wr_pad):
    return pl.pallas_call(
        _plan_body,
        grid=(1,),
        in_specs=[
            pl.BlockSpec((H, S // 2, DH), lambda i: (0, 0, 0)),
            pl.BlockSpec((H, S // 2, DH), lambda i: (0, 0, 0)),
            pl.BlockSpec((H * DH, D), lambda i: (0, 0)),
            pl.BlockSpec((S, D), lambda i: (0, 0)),
            pl.BlockSpec((1, D), lambda i: (0, 0)),
            pl.BlockSpec((D, 128), lambda i: (0, 0)),
        ],
        out_specs=[
            pl.BlockSpec((S, D), lambda i: (0, 0)),
            pl.BlockSpec((S, D), lambda i: (0, 0)),
            pl.BlockSpec((S, 1), lambda i: (0, 0)),
            pl.BlockSpec((S, 1), lambda i: (0, 0)),
            pl.BlockSpec((S, 1), lambda i: (0, 0)),
            pl.BlockSpec((S, 1), lambda i: (0, 0)),
            pl.BlockSpec((128, 1), lambda i: (0, 0)),
            pl.BlockSpec((128, 1), lambda i: (0, 0)),
        ],
        out_shape=[
            jax.ShapeDtypeStruct((S, D), jnp.float32),
            jax.ShapeDtypeStruct((S, D), jnp.float32),
            jax.ShapeDtypeStruct((S, 1), jnp.int32),
            jax.ShapeDtypeStruct((S, 1), jnp.int32),
            jax.ShapeDtypeStruct((S, 1), jnp.float32),
            jax.ShapeDtypeStruct((S, 1), jnp.float32),
            jax.ShapeDtypeStruct((128, 1), jnp.int32),
            jax.ShapeDtypeStruct((128, 1), jnp.int32),
        ],
        scratch_shapes=[pltpu.VMEM((S, H * DH), jnp.bfloat16)],
        interpret=_INTERPRET,
    )(oA, oB, Wo2d, res, ln2, Wr_pad)


_NW = 32
_CHUNK = S // _NW


def _dispatch_body(x2_hbm, posA_hbm, posB_hbm, xs_hbm,
                   idxA_v, idxB_v, rows_v, semA, semB):
    wid = jax.lax.axis_index("s") * 2 + jax.lax.axis_index("c")
    base = wid * _CHUNK
    pltpu.sync_copy(posA_hbm.at[pl.ds(base, _CHUNK)], idxA_v)
    pltpu.sync_copy(posB_hbm.at[pl.ds(base, _CHUNK)], idxB_v)
    pltpu.sync_copy(x2_hbm.at[pl.ds(base, _CHUNK)], rows_v)
    cpA = pltpu.async_copy(rows_v, xs_hbm.at[idxA_v], semA)
    cpB = pltpu.async_copy(rows_v, xs_hbm.at[idxB_v], semB)
    cpA.wait()
    cpB.wait()


def _dispatch_call(x2, posA, posB):
    mesh = plsc.VectorSubcoreMesh(core_axis_name="c", subcore_axis_name="s")
    return pl.kernel(
        _dispatch_body,
        mesh=mesh,
        out_type=jax.ShapeDtypeStruct((XROWS, D), jnp.float32),
        scratch_types=[
            pltpu.VMEM((_CHUNK,), jnp.int32),
            pltpu.VMEM((_CHUNK,), jnp.int32),
            pltpu.VMEM((_CHUNK, D), jnp.float32),
            pltpu.SemaphoreType.DMA,
            pltpu.SemaphoreType.DMA,
        ],
    )(x2, posA, posB)


def _gmm_body(te_ref, tv_ref, x_ref, wg_ref, wu_ref, wd_ref, y_ref):
    i = pl.program_id(0)

    @pl.when(tv_ref[i] == 1)
    def _():
        x = x_ref[...]
        px = jax.lax.Precision.DEFAULT
        g = jnp.dot(x, wg_ref[0], preferred_element_type=jnp.float32,
                    precision=px)
        u = jnp.dot(x, wu_ref[0], preferred_element_type=jnp.float32,
                    precision=px)
        a = g * jax.nn.sigmoid(g) * u
        y_ref[...] = jnp.dot(a, wd_ref[0], preferred_element_type=jnp.float32,
                             precision=px)


def _gmm_call(te, tv, xs, Wg, Wu, Wd):
    grid_spec = pltpu.PrefetchScalarGridSpec(
        num_scalar_prefetch=2,
        grid=(NT,),
        in_specs=[
            pl.BlockSpec((M, D), lambda i, te, tv: (i, 0)),
            pl.BlockSpec((1, D, F), lambda i, te, tv: (te[i], 0, 0)),
            pl.BlockSpec((1, D, F), lambda i, te, tv: (te[i], 0, 0)),
            pl.BlockSpec((1, F, D), lambda i, te, tv: (te[i], 0, 0)),
        ],
        out_specs=pl.BlockSpec((M, D), lambda i, te, tv: (i, 0)),
    )
    return pl.pallas_call(
        _gmm_body,
        grid_spec=grid_spec,
        out_shape=jax.ShapeDtypeStruct((XROWS, D), jnp.float32),
        interpret=_INTERPRET,
    )(te, tv, xs, Wg, Wu, Wd)


def _mgather_body(ys_hbm, posA_hbm, posB_hbm, yA_hbm, yB_hbm,
                  idx_v, buf_v, sem):
    wid = jax.lax.axis_index("s") * 2 + jax.lax.axis_index("c")
    base = wid * _CHUNK
    pltpu.sync_copy(posA_hbm.at[pl.ds(base, _CHUNK)], idx_v)
    pltpu.async_copy(ys_hbm.at[idx_v], buf_v, sem).wait()
    pltpu.sync_copy(buf_v, yA_hbm.at[pl.ds(base, _CHUNK)])
    pltpu.sync_copy(posB_hbm.at[pl.ds(base, _CHUNK)], idx_v)
    pltpu.async_copy(ys_hbm.at[idx_v], buf_v, sem).wait()
    pltpu.sync_copy(buf_v, yB_hbm.at[pl.ds(base, _CHUNK)])


def _mgather_call(ys, posA, posB):
    mesh = plsc.VectorSubcoreMesh(core_axis_name="c", subcore_axis_name="s")
    return pl.kernel(
        _mgather_body,
        mesh=mesh,
        out_type=[
            jax.ShapeDtypeStruct((S, D), jnp.float32),
            jax.ShapeDtypeStruct((S, D), jnp.float32),
        ],
        scratch_types=[
            pltpu.VMEM((_CHUNK,), jnp.int32),
            pltpu.VMEM((_CHUNK, D), jnp.float32),
            pltpu.SemaphoreType.DMA,
        ],
    )(ys, posA, posB)


def _mix_body(h_ref, yA_ref, yB_ref, wA_ref, wB_ref, o_ref):
    o_ref[...] = h_ref[...] + RES_SCALE * (
        wA_ref[...] * yA_ref[...] + wB_ref[...] * yB_ref[...])


def _mix_call(h, yA, yB, wA, wB):
    return pl.pallas_call(
        _mix_body,
        grid=(NSB,),
        in_specs=[
            pl.BlockSpec((SB, D), lambda i: (i, 0)),
            pl.BlockSpec((SB, D), lambda i: (i, 0)),
            pl.BlockSpec((SB, D), lambda i: (i, 0)),
            pl.BlockSpec((SB, 1), lambda i: (i, 0)),
            pl.BlockSpec((SB, 1), lambda i: (i, 0)),
        ],
        out_specs=pl.BlockSpec((SB, D), lambda i: (i, 0)),
        out_shape=jax.ShapeDtypeStruct((S, D), jnp.float32),
        interpret=_INTERPRET,
    )(h, yA, yB, wA, wB)


@jax.jit
def _run(hidden_states, position_ids, ln1_w, ln2_w,
         Wq, Wk, Wv, Wo, Wr, Wg, Wu, Wd):
    x = hidden_states.reshape(S, D)
    pos = position_ids.reshape(S, 1)
    ln1 = ln1_w.reshape(1, D)
    ln2 = ln2_w.reshape(1, D)
    Wr_pad = jnp.pad(Wr, ((0, 0), (0, 128 - E)))

    bf = jnp.bfloat16
    q, k, v = _qkv_call(pos, x, ln1, Wq.astype(bf), Wk.astype(bf),
                        Wv.astype(bf))
    oA, oB = _attn_call(q, k, v)
    h, x2, posA, posB, wA, wB, te, tv = _plan_call(
        oA, oB, Wo.astype(bf), x, ln2, Wr_pad)
    te = te.reshape(128)
    tv = tv.reshape(128)
    posAf = posA.reshape(S)
    posBf = posB.reshape(S)
    xs = _dispatch_call(x2, posAf, posBf)
    ys = _gmm_call(te, tv, xs, Wg, Wu, Wd)
    yA, yB = _mgather_call(ys, posAf, posBf)
    out = _mix_call(h, yA, yB, wA, wB)
    return out.reshape(B, S, D)


def kernel(hidden_states, position_ids, ln1_w, ln2_w,
           Wq, Wk, Wv, Wo, Wr, Wg, Wu, Wd):
    return _run(hidden_states, position_ids, ln1_w, ln2_w,
                Wq, Wk, Wv, Wo, Wr, Wg, Wu, Wd)
